# baseline (speedup 1.0000x reference)
import functools

import jax
import jax.numpy as jnp
from jax import lax
from jax.experimental import pallas as pl
from jax.experimental.pallas import tpu as pltpu
from jax.experimental.pallas import tpu_sc as plsc

EPS_ = 1e-08
B_, C_ = 16384, 1000
NC_, NS_, L_ = 2, 16, 16
NW_ = NC_ * NS_
HSC_ = 6144
RPW_ = HSC_ // NW_
R_ = 16
NCH_ = RPW_ // R_
NV_ = C_ // L_
TAIL_ = C_ - L_
CP_ = 1024


def _sc_body(x_hbm, y_hbm, w_hbm, out_hbm,
             idx_v, xb, wb, part_v, sx0, sx1, sw0, sw1):
    wid = lax.axis_index("s") * NC_ + lax.axis_index("c")
    base = wid * RPW_
    pltpu.sync_copy(y_hbm.at[pl.ds(base, RPW_)], idx_v)
    sx = (sx0, sx1)
    sw = (sw0, sw1)
    tail_mask = jnp.arange(L_, dtype=jnp.int32) >= (L_ - (C_ - NV_ * L_))

    def start(chunk, b):
        row0 = base + pl.multiple_of(chunk * R_, 8)
        pltpu.async_copy(x_hbm.at[pl.ds(row0, R_)], xb.at[b], sx[b])
        i0 = pl.multiple_of(chunk * R_, 8)
        pltpu.async_copy(w_hbm.at[idx_v.at[pl.ds(i0, R_)]], wb.at[b], sw[b])

    def wait(b):
        pltpu.make_async_copy(x_hbm.at[pl.ds(0, R_)], xb.at[b], sx[b]).wait()
        pltpu.make_async_copy(
            w_hbm.at[idx_v.at[pl.ds(0, R_)]], wb.at[b], sw[b]).wait()

    def compute(chunk, b):
        def row(r, carry):
            acc = xb[b, r, pl.ds(0, L_)] * wb[b, r, pl.ds(0, L_)]
            for i in range(1, NV_):
                acc += xb[b, r, pl.ds(i * L_, L_)] * wb[b, r, pl.ds(i * L_, L_)]
            xt = xb[b, r, pl.ds(TAIL_, L_)]
            wt = wb[b, r, pl.ds(TAIL_, L_)]
            acc = acc + jnp.where(tail_mask, xt * wt, jnp.float32(0.0))
            part_v[pl.ds((chunk * R_ + r) * L_, L_)] = acc
            return carry

        lax.fori_loop(0, R_, row, 0)

    start(0, 0)

    def outer(j, carry):
        c0 = j * 2
        start(c0 + 1, 1)
        wait(0)
        compute(c0, 0)

        @pl.when(c0 + 2 < NCH_)
        def _():
            start(c0 + 2, 0)

        wait(1)
        compute(c0 + 1, 1)
        return carry

    lax.fori_loop(0, NCH_ // 2, outer, 0)
    pltpu.sync_copy(part_v, out_hbm.at[pl.ds(base * L_, RPW_ * L_)])


_sc_call = pl.kernel(
    _sc_body,
    out_type=jax.ShapeDtypeStruct((HSC_ * L_,), jnp.float32),
    mesh=plsc.VectorSubcoreMesh(
        core_axis_name="c", subcore_axis_name="s",
        num_cores=NC_, num_subcores=NS_),
    scratch_types=[
        pltpu.VMEM((RPW_,), jnp.int32),
        pltpu.VMEM((2, R_, C_), jnp.float32),
        pltpu.VMEM((2, R_, CP_), jnp.float32),
        pltpu.VMEM((RPW_ * L_,), jnp.float32),
        pltpu.SemaphoreType.DMA,
        pltpu.SemaphoreType.DMA,
        pltpu.SemaphoreType.DMA,
        pltpu.SemaphoreType.DMA,
    ],
)

BLK_ = 1024


def _tc_body(y_ref, x_ref, w_ref, out_ref):
    i = pl.program_id(0)
    y_col = y_ref[0]
    classes = jax.lax.broadcasted_iota(jnp.int32, (BLK_, C_), 1)
    onehot = (y_col == classes).astype(jnp.bfloat16)
    w_b = w_ref[...].astype(jnp.bfloat16)
    wy = jax.lax.dot_general(
        onehot, w_b, (((1,), (0,)), ((), ())),
        preferred_element_type=jnp.float32)
    s = jnp.sum(wy * x_ref[...], axis=1, keepdims=True)
    part = jnp.sum(-jnp.log(s + EPS_)).reshape(1, 1)

    @pl.when(i == 0)
    def _():
        out_ref[...] = jnp.zeros((1, 1), jnp.float32)

    out_ref[...] += part


def _fin_body(p_ref, t_ref, out_ref):
    blk = p_ref[...]
    acc = None
    for g in range(8):
        rs = jnp.sum(blk[:, g * L_:(g + 1) * L_], axis=1, keepdims=True)
        lg = -jnp.log(rs + EPS_)
        acc = lg if acc is None else acc + lg
    out_ref[...] = jnp.sum(acc).reshape(1, 1) + t_ref[...]


@jax.jit
def kernel(x, y, w):
    y32 = y.astype(jnp.int32)
    w_pad = jnp.pad(w, ((0, 0), (0, CP_ - C_)))
    part = _sc_call(x, y32, w_pad)

    ntc = (B_ - HSC_) // BLK_
    off = HSC_ // BLK_
    y3 = y32.reshape(B_ // BLK_, BLK_, 1)
    tc_tot = pl.pallas_call(
        _tc_body,
        grid=(ntc,),
        in_specs=[
            pl.BlockSpec((1, BLK_, 1), lambda i: (i + off, 0, 0)),
            pl.BlockSpec((BLK_, C_), lambda i: (i + off, 0)),
            pl.BlockSpec((C_, C_), lambda i: (0, 0)),
        ],
        out_specs=pl.BlockSpec((1, 1), lambda i: (0, 0)),
        out_shape=jax.ShapeDtypeStruct((1, 1), jnp.float32),
    )(y3, x, w)

    total = pl.pallas_call(
        _fin_body,
        in_specs=[
            pl.BlockSpec((HSC_ // 8, 128), lambda: (0, 0)),
            pl.BlockSpec((1, 1), lambda: (0, 0)),
        ],
        out_specs=pl.BlockSpec((1, 1), lambda: (0, 0)),
        out_shape=jax.ShapeDtypeStruct((1, 1), jnp.float32),
    )(part.reshape(HSC_ // 8, 128), tc_tot)
    return total[0, 0] / B_


# speedup vs baseline: 2.9106x; 2.9106x over previous
import functools

import jax
import jax.numpy as jnp
from jax import lax
from jax.experimental import pallas as pl
from jax.experimental.pallas import tpu as pltpu
from jax.experimental.pallas import tpu_sc as plsc

EPS_ = 1e-08
B_, C_ = 16384, 1000
BLK_ = 2048


def _tc_body(y_ref, xt_ref, w_ref, out_ref):
    i = pl.program_id(0)
    y_row = y_ref[0]
    classes = jax.lax.broadcasted_iota(jnp.int32, (C_, BLK_), 0)
    onehot_t = (classes == y_row).astype(jnp.bfloat16)
    w_b = w_ref[...].astype(jnp.bfloat16)
    wy_t = jax.lax.dot_general(
        w_b, onehot_t, (((0,), (0,)), ((), ())),
        preferred_element_type=jnp.float32)
    s = jnp.sum(wy_t * xt_ref[...], axis=0, keepdims=True)
    part = jnp.sum(-jnp.log(s + EPS_)).reshape(1, 1)

    @pl.when(i == 0)
    def _():
        out_ref[...] = jnp.zeros((1, 1), jnp.float32)

    out_ref[...] += part


@jax.jit
def kernel(x, y, w):
    y32 = y.astype(jnp.int32)
    xt = x.T
    nblk = B_ // BLK_
    y3 = y32.reshape(nblk, 1, BLK_)
    total = pl.pallas_call(
        _tc_body,
        grid=(nblk,),
        in_specs=[
            pl.BlockSpec((1, 1, BLK_), lambda i: (i, 0, 0)),
            pl.BlockSpec((C_, BLK_), lambda i: (0, i)),
            pl.BlockSpec((C_, C_), lambda i: (0, 0)),
        ],
        out_specs=pl.BlockSpec((1, 1), lambda i: (0, 0)),
        out_shape=jax.ShapeDtypeStruct((1, 1), jnp.float32),
    )(y3, xt, w)
    return total[0, 0] / B_
